# baseline (speedup 1.0000x reference)
import functools

import jax
import jax.numpy as jnp
from jax import lax
from jax.experimental import pallas as pl
from jax.experimental.pallas import tpu as pltpu
from jax.experimental.pallas import tpu_sc as plsc

N_NODES = 10000
N_EDGES = 160000
IN_DIM = 256
OUT_DIM = 256

NC = 2
NS = 16
HALF = IN_DIM // 2
E_PER_TILE = N_EDGES // NS
CH = 64
NCHUNK = 160
E_TILE_PAD = NCHUNK * CH
E_PAD = NS * E_TILE_PAD
DUMP_ROW = 10008
NPAD = 10112
ROWS_PER_TILE = NPAD // NS


def _sc_segment_sum(xs, gidx, dstc, zeros):
  mesh = plsc.VectorSubcoreMesh(core_axis_name="c", subcore_axis_name="s")

  @functools.partial(
      pl.kernel,
      out_type=(
          jax.ShapeDtypeStruct((NPAD, HALF), jnp.float32),
          jax.ShapeDtypeStruct((NPAD, HALF), jnp.float32),
      ),
      mesh=mesh,
      scratch_types=[
          pltpu.VMEM_SHARED((NPAD, HALF), jnp.float32),
          pltpu.VMEM((E_TILE_PAD,), jnp.int32),
          pltpu.VMEM((NCHUNK, CH), jnp.int32),
          pltpu.VMEM((CH, HALF), jnp.float32),
          pltpu.VMEM((CH, HALF), jnp.float32),
          pltpu.SemaphoreType.DMA,
          pltpu.SemaphoreType.DMA,
      ],
  )
  def k(xs_hbm, gidx_hbm, dstc_hbm, zeros_hbm, agg0_hbm, agg1_hbm,
        aggs, idxv, dstv, rows0, rows1, sem0, sem1):
    c = lax.axis_index("c")
    s = lax.axis_index("s")
    base = s * ROWS_PER_TILE

    pltpu.sync_copy(zeros_hbm, aggs.at[pl.ds(base, ROWS_PER_TILE)])
    pltpu.sync_copy(gidx_hbm.at[c, s], idxv)
    pltpu.sync_copy(dstc_hbm.at[s], dstv)
    plsc.subcore_barrier()

    def gather(i, buf, sem):
      pltpu.async_copy(xs_hbm.at[idxv.at[pl.ds(i * CH, CH)]], buf, sem)

    def wait_scatter(i, buf, sem):
      pltpu.make_async_copy(xs_hbm.at[idxv.at[pl.ds(i * CH, CH)]], buf,
                            sem).wait()
      pltpu.sync_copy(buf, aggs.at[dstv.at[i]], add=True)

    gather(0, rows0, sem0)

    def pair(g, carry):
      i = 2 * g + 1
      gather(i, rows1, sem1)
      wait_scatter(i - 1, rows0, sem0)
      gather(i + 1, rows0, sem0)
      wait_scatter(i, rows1, sem1)
      return carry

    lax.fori_loop(0, NCHUNK // 2 - 1, pair, 0)
    gather(NCHUNK - 1, rows1, sem1)
    wait_scatter(NCHUNK - 2, rows0, sem0)
    wait_scatter(NCHUNK - 1, rows1, sem1)
    plsc.subcore_barrier()

    @pl.when(c == 0)
    def _():
      pltpu.sync_copy(aggs.at[pl.ds(base, ROWS_PER_TILE)],
                      agg0_hbm.at[pl.ds(base, ROWS_PER_TILE)])

    @pl.when(c == 1)
    def _():
      pltpu.sync_copy(aggs.at[pl.ds(base, ROWS_PER_TILE)],
                      agg1_hbm.at[pl.ds(base, ROWS_PER_TILE)])

  return k(xs, gidx, dstc, zeros)


def _xw_body(xr, wr, b, o):
  o[...] = jnp.dot(xr[...], wr[...], preferred_element_type=jnp.float32) + b[...]


def _tc_xwroot(x, W_root, b_rel):
  BM = 1000
  return pl.pallas_call(
      _xw_body,
      grid=(N_NODES // BM,),
      in_specs=[
          pl.BlockSpec((BM, IN_DIM), lambda i: (i, 0)),
          pl.BlockSpec((IN_DIM, OUT_DIM), lambda i: (0, 0)),
          pl.BlockSpec((1, OUT_DIM), lambda i: (0, 0)),
      ],
      out_specs=pl.BlockSpec((BM, OUT_DIM), lambda i: (i, 0)),
      out_shape=jax.ShapeDtypeStruct((N_NODES, OUT_DIM), jnp.float32),
  )(x, W_root, b_rel.reshape(1, -1))


def _mm_body(a0, a1, xw, w0, w1, o):
  acc = jnp.dot(a0[...], w0[...], preferred_element_type=jnp.float32)
  acc += jnp.dot(a1[...], w1[...], preferred_element_type=jnp.float32)
  o[...] = acc + xw[...]


def _tc_linear(agg0, agg1, xw, W_rel):
  BM = 1000
  return pl.pallas_call(
      _mm_body,
      grid=(N_NODES // BM,),
      in_specs=[
          pl.BlockSpec((BM, HALF), lambda i: (i, 0)),
          pl.BlockSpec((BM, HALF), lambda i: (i, 0)),
          pl.BlockSpec((BM, OUT_DIM), lambda i: (i, 0)),
          pl.BlockSpec((HALF, OUT_DIM), lambda i: (0, 0)),
          pl.BlockSpec((HALF, OUT_DIM), lambda i: (0, 0)),
      ],
      out_specs=pl.BlockSpec((BM, OUT_DIM), lambda i: (i, 0)),
      out_shape=jax.ShapeDtypeStruct((N_NODES, OUT_DIM), jnp.float32),
  )(agg0, agg1, xw, W_rel[:HALF], W_rel[HALF:])


@jax.jit
def kernel(x, edge_index, W_rel, b_rel, W_root):
  src = edge_index[0].astype(jnp.int32)
  dst = edge_index[1].astype(jnp.int32)

  xs = jnp.concatenate([x[:, :HALF], x[:, HALF:]], axis=0)
  npad_e = E_PAD - N_EDGES
  src_p = jnp.concatenate([src, jnp.zeros((npad_e,), jnp.int32)])
  dst_p = jnp.concatenate([dst, jnp.full((npad_e,), DUMP_ROW, jnp.int32)])
  gidx = jnp.stack([src_p, src_p + N_NODES]).reshape(NC, NS, E_TILE_PAD)
  dstc = dst_p.reshape(NS, NCHUNK, CH)
  zeros = jnp.zeros((ROWS_PER_TILE, HALF), jnp.float32)

  xw = _tc_xwroot(x, W_root, b_rel)
  agg0, agg1 = _sc_segment_sum(xs, gidx, dstc, zeros)
  return _tc_linear(agg0, agg1, xw, W_rel)


# speedup vs baseline: 2.0548x; 2.0548x over previous
import functools

import jax
import jax.numpy as jnp
from jax import lax
from jax.experimental import pallas as pl
from jax.experimental.pallas import tpu as pltpu
from jax.experimental.pallas import tpu_sc as plsc

N_NODES = 10000
N_EDGES = 160000
IN_DIM = 256
OUT_DIM = 256

NC = 2
NS = 16
HALF = IN_DIM // 2
E_PER_TILE = N_EDGES // NS
CH = 80
NCHUNK = E_PER_TILE // CH
NPAD = 10112
ROWS_PER_TILE = NPAD // NS


def _sc_segment_sum(xs, gidx, dstc, zeros):
  mesh = plsc.VectorSubcoreMesh(core_axis_name="c", subcore_axis_name="s")

  @functools.partial(
      pl.kernel,
      out_type=(
          jax.ShapeDtypeStruct((NPAD, HALF), jnp.float32),
          jax.ShapeDtypeStruct((NPAD, HALF), jnp.float32),
      ),
      mesh=mesh,
      scratch_types=[
          pltpu.VMEM_SHARED((NPAD, HALF), jnp.float32),
          pltpu.VMEM((E_PER_TILE,), jnp.int32),
          pltpu.VMEM((NCHUNK, CH), jnp.int32),
          pltpu.VMEM((CH, HALF), jnp.float32),
          pltpu.VMEM((CH, HALF), jnp.float32),
          pltpu.SemaphoreType.DMA,
          pltpu.SemaphoreType.DMA,
      ],
  )
  def k(xs_hbm, gidx_hbm, dstc_hbm, zeros_hbm, agg0_hbm, agg1_hbm,
        aggs, idxv, dstv, rows0, rows1, sem0, sem1):
    c = lax.axis_index("c")
    s = lax.axis_index("s")
    base = s * ROWS_PER_TILE

    pltpu.sync_copy(zeros_hbm, aggs.at[pl.ds(base, ROWS_PER_TILE)])
    pltpu.sync_copy(gidx_hbm.at[c, s], idxv)
    pltpu.sync_copy(dstc_hbm.at[s], dstv)
    plsc.subcore_barrier()

    def gather(i, buf, sem):
      pltpu.async_copy(xs_hbm.at[idxv.at[pl.ds(i * CH, CH)]], buf, sem)

    def wait_scatter(i, buf, sem):
      pltpu.make_async_copy(xs_hbm.at[idxv.at[pl.ds(i * CH, CH)]], buf,
                            sem).wait()
      pltpu.sync_copy(buf, aggs.at[dstv.at[i]], add=True)

    gather(0, rows0, sem0)

    def pair(g, carry):
      i = 2 * g + 1
      gather(i, rows1, sem1)
      wait_scatter(i - 1, rows0, sem0)
      gather(i + 1, rows0, sem0)
      wait_scatter(i, rows1, sem1)
      return carry

    lax.fori_loop(0, (NCHUNK - 1) // 2, pair, 0)
    wait_scatter(NCHUNK - 1, rows0, sem0)
    plsc.subcore_barrier()

    @pl.when(c == 0)
    def _():
      pltpu.sync_copy(aggs.at[pl.ds(base, ROWS_PER_TILE)],
                      agg0_hbm.at[pl.ds(base, ROWS_PER_TILE)])

    @pl.when(c == 1)
    def _():
      pltpu.sync_copy(aggs.at[pl.ds(base, ROWS_PER_TILE)],
                      agg1_hbm.at[pl.ds(base, ROWS_PER_TILE)])

  return k(xs, gidx, dstc, zeros)


def _xw_body(xr, wr, b, o):
  o[...] = jnp.dot(xr[...], wr[...], preferred_element_type=jnp.float32) + b[...]


def _tc_xwroot(x, W_root, b_rel):
  BM = 1000
  return pl.pallas_call(
      _xw_body,
      grid=(N_NODES // BM,),
      in_specs=[
          pl.BlockSpec((BM, IN_DIM), lambda i: (i, 0)),
          pl.BlockSpec((IN_DIM, OUT_DIM), lambda i: (0, 0)),
          pl.BlockSpec((1, OUT_DIM), lambda i: (0, 0)),
      ],
      out_specs=pl.BlockSpec((BM, OUT_DIM), lambda i: (i, 0)),
      out_shape=jax.ShapeDtypeStruct((N_NODES, OUT_DIM), jnp.float32),
  )(x, W_root, b_rel.reshape(1, -1))


def _mm_body(a0, a1, xw, w0, w1, o):
  acc = jnp.dot(a0[...], w0[...], preferred_element_type=jnp.float32)
  acc += jnp.dot(a1[...], w1[...], preferred_element_type=jnp.float32)
  o[...] = acc + xw[...]


def _tc_linear(agg0, agg1, xw, W_rel):
  BM = 1000
  return pl.pallas_call(
      _mm_body,
      grid=(N_NODES // BM,),
      in_specs=[
          pl.BlockSpec((BM, HALF), lambda i: (i, 0)),
          pl.BlockSpec((BM, HALF), lambda i: (i, 0)),
          pl.BlockSpec((BM, OUT_DIM), lambda i: (i, 0)),
          pl.BlockSpec((HALF, OUT_DIM), lambda i: (0, 0)),
          pl.BlockSpec((HALF, OUT_DIM), lambda i: (0, 0)),
      ],
      out_specs=pl.BlockSpec((BM, OUT_DIM), lambda i: (i, 0)),
      out_shape=jax.ShapeDtypeStruct((N_NODES, OUT_DIM), jnp.float32),
  )(agg0, agg1, xw, W_rel[:HALF], W_rel[HALF:])


@jax.jit
def kernel(x, edge_index, W_rel, b_rel, W_root):
  src = edge_index[0].astype(jnp.int32)
  dst = edge_index[1].astype(jnp.int32)

  xs = jnp.concatenate([x[:, :HALF], x[:, HALF:]], axis=0)
  gidx = jnp.stack([src, src + N_NODES]).reshape(NC, NS, E_PER_TILE)
  dstc = dst.reshape(NS, NCHUNK, CH)
  zeros = jnp.zeros((ROWS_PER_TILE, HALF), jnp.float32)

  xw = _tc_xwroot(x, W_root, b_rel)
  agg0, agg1 = _sc_segment_sum(xs, gidx, dstc, zeros)
  return _tc_linear(agg0, agg1, xw, W_rel)
